# baseline (speedup 1.0000x reference)
import functools

import jax
import jax.numpy as jnp
from jax import lax
from jax.experimental import pallas as pl
from jax.experimental.pallas import tpu as pltpu
from jax.experimental.pallas import tpu_sc as plsc

_B = 16384
_F = 32
_CHUNK = 128


@functools.cache
def _build():
    info = plsc.get_sparse_core_info()
    nc, ns, nl = info.num_cores, info.num_subcores, info.num_lanes
    nw = nc * ns
    bpw = _B // nw
    nch = bpw // _CHUNK
    mesh = plsc.VectorSubcoreMesh(core_axis_name="c", subcore_axis_name="s")

    @functools.partial(
        pl.kernel,
        mesh=mesh,
        out_type=jax.ShapeDtypeStruct((_B,), jnp.float32),
        compiler_params=pltpu.CompilerParams(needs_layout_passes=False,
                                             use_tc_tiling_on_sc=False),
        scratch_types=[
            pltpu.VMEM((nch, _CHUNK), jnp.int32),
            pltpu.VMEM((nch, _CHUNK), jnp.int32),
            pltpu.VMEM((bpw, _F), jnp.float32),
            pltpu.VMEM((bpw, _F), jnp.float32),
            pltpu.VMEM((bpw,), jnp.float32),
            pltpu.SemaphoreType.DMA,
            pltpu.SemaphoreType.DMA,
        ],
    )
    def sc_dot(uidx_h, iidx_h, utab_h, itab_h, out_h,
               uidx_v, iidx_v, urows_v, irows_v, out_v, sem_u, sem_i):
        wid = lax.axis_index("s") * nc + lax.axis_index("c")
        base = wid * bpw

        for j in range(nch):
            pltpu.sync_copy(uidx_h.at[pl.ds(base + j * _CHUNK, _CHUNK)],
                            uidx_v.at[j])
            pltpu.sync_copy(iidx_h.at[pl.ds(base + j * _CHUNK, _CHUNK)],
                            iidx_v.at[j])

        copies = []
        for j in range(nch):
            copies.append(pltpu.async_copy(
                utab_h.at[uidx_v.at[j]],
                urows_v.at[pl.ds(j * _CHUNK, _CHUNK)], sem_u))
            copies.append(pltpu.async_copy(
                itab_h.at[iidx_v.at[j]],
                irows_v.at[pl.ds(j * _CHUNK, _CHUNK)], sem_i))
        for cp in copies:
            cp.wait()

        lanes = lax.iota(jnp.int32, nl)

        def body(g, carry):
            rows = g * nl + lanes
            acc = jnp.zeros((nl,), jnp.float32)
            for f in range(_F):
                col = jnp.full((nl,), f, jnp.int32)
                u = plsc.load_gather(urows_v, [rows, col])
                t = plsc.load_gather(irows_v, [rows, col])
                acc = acc + u * t
            out_v[pl.ds(g * nl, nl)] = acc
            return carry

        lax.fori_loop(0, bpw // nl, body, 0)
        pltpu.sync_copy(out_v, out_h.at[pl.ds(base, bpw)])

    return sc_dot


def kernel(user_indices, item_indices, user_table, item_table):
    sc_dot = _build()
    return sc_dot(user_indices.astype(jnp.int32),
                  item_indices.astype(jnp.int32),
                  user_table, item_table)
